# baseline (speedup 1.0000x reference)
import functools

import jax
import jax.numpy as jnp
import numpy as np
from jax import lax
from jax.experimental import pallas as pl
from jax.experimental.pallas import tpu as pltpu
from jax.experimental.pallas import tpu_sc as plsc

_N = 10000
_E = 320000
_H = 128
_L = 4
_NCENT = 32
_CUT = 6.0
_GAMMA = 10.0 / (_CUT - 0.0 + 1e-06) ** 2

_NSUB = 16
_CHUNK = 128
_CPW = 80
_EPW = _CHUNK * _CPW
_CAP = _NSUB * _EPW
_ET = 2 * _CAP
_NH = _N // 2
_NPC = 5120
_RPS = _NPC // _NSUB

_EBLK = 2048
_NBLK = 2000
_MBLK = 1000


def _edge_mlp_body(d_ref, w1_ref, b1_ref, w2_ref, b2_ref, out_ref):
    centers = lax.broadcasted_iota(jnp.int32, (1, _NCENT), 1).astype(jnp.float32) * (
        _CUT / (_NCENT - 1))
    diff = d_ref[...] - centers
    rbf = jnp.exp((-_GAMMA) * diff * diff)
    h = jnp.dot(rbf, w1_ref[...], preferred_element_type=jnp.float32)
    h = h + b1_ref[...]
    h = h * jax.nn.sigmoid(h)
    e = jnp.dot(h, w2_ref[...], preferred_element_type=jnp.float32)
    out_ref[...] = e + b2_ref[...]


def _edge_mlp(d, w1, b1, w2, b2):
    return pl.pallas_call(
        _edge_mlp_body,
        grid=(_ET // _EBLK,),
        in_specs=[
            pl.BlockSpec((_EBLK, 1), lambda i: (i, 0)),
            pl.BlockSpec((_NCENT, _H), lambda i: (0, 0)),
            pl.BlockSpec((1, _H), lambda i: (0, 0)),
            pl.BlockSpec((_H, _H), lambda i: (0, 0)),
            pl.BlockSpec((1, _H), lambda i: (0, 0)),
        ],
        out_specs=pl.BlockSpec((_EBLK, _H), lambda i: (i, 0)),
        out_shape=jax.ShapeDtypeStruct((_ET, _H), jnp.float32),
    )(d, w1, b1, w2, b2)


def _embed_body(z_ref, emb_ref, out_ref):
    ids = lax.broadcasted_iota(jnp.int32, (_NBLK, _H), 1)
    oh = (z_ref[...] == ids).astype(jnp.float32)
    out_ref[...] = jnp.dot(oh, emb_ref[...], preferred_element_type=jnp.float32)


def _embed(z2d, emb_pad):
    return pl.pallas_call(
        _embed_body,
        grid=(_N // _NBLK,),
        in_specs=[
            pl.BlockSpec((_NBLK, 1), lambda i: (i, 0)),
            pl.BlockSpec((_H, _H), lambda i: (0, 0)),
        ],
        out_specs=pl.BlockSpec((_NBLK, _H), lambda i: (i, 0)),
        out_shape=jax.ShapeDtypeStruct((_N, _H), jnp.float32),
    )(z2d, emb_pad)


def _node_mlp_body(x_ref, a_ref, w1_ref, b1_ref, w2_ref, b2_ref, out_ref):
    h = x_ref[...] + a_ref[0]
    t = jnp.dot(h, w1_ref[...], preferred_element_type=jnp.float32) + b1_ref[...]
    t = t * jax.nn.sigmoid(t)
    o = jnp.dot(t, w2_ref[...], preferred_element_type=jnp.float32) + b2_ref[...]
    out_ref[...] = o * jax.nn.sigmoid(o)


def _node_mlp(x, agg, w1, b1, w2, b2):
    nhb = _NH // _MBLK
    return pl.pallas_call(
        _node_mlp_body,
        grid=(_N // _MBLK,),
        in_specs=[
            pl.BlockSpec((_MBLK, _H), lambda i: (i, 0)),
            pl.BlockSpec((1, _MBLK, _H), lambda i: (i // nhb, i % nhb, 0)),
            pl.BlockSpec((_H, _H), lambda i: (0, 0)),
            pl.BlockSpec((1, _H), lambda i: (0, 0)),
            pl.BlockSpec((_H, _H), lambda i: (0, 0)),
            pl.BlockSpec((1, _H), lambda i: (0, 0)),
        ],
        out_specs=pl.BlockSpec((_MBLK, _H), lambda i: (i, 0)),
        out_shape=jax.ShapeDtypeStruct((_N, _H), jnp.float32),
    )(x, agg, w1, b1, w2, b2)


def _head_body(x_ref, wp1_ref, bp1_ref, wp2_ref, bp2_ref, out_ref):
    g = jnp.mean(x_ref[...], axis=0, keepdims=True)
    t = jnp.dot(g, wp1_ref[...], preferred_element_type=jnp.float32) + bp1_ref[...]
    t = t * jax.nn.sigmoid(t)
    zz = jnp.dot(t, wp2_ref[...], preferred_element_type=jnp.float32) + bp2_ref[...]
    nrm = jnp.sqrt(jnp.sum(zz * zz, axis=-1, keepdims=True))
    out_ref[...] = zz / jnp.maximum(nrm, 1e-12)


def _head(x, wp1, bp1, wp2, bp2):
    return pl.pallas_call(
        _head_body,
        out_shape=jax.ShapeDtypeStruct((1, _H), jnp.float32),
    )(x, wp1, bp1, wp2, bp2)


_SC_MESH = plsc.VectorSubcoreMesh(core_axis_name="c", subcore_axis_name="s")


@functools.partial(
    pl.kernel,
    out_type=jax.ShapeDtypeStruct((2, _NPC, _H), jnp.float32),
    mesh=_SC_MESH,
    scratch_types=[
        pltpu.VMEM((4, _CHUNK), jnp.int32),
        pltpu.VMEM((4, _CHUNK), jnp.int32),
        pltpu.VMEM((3, _CHUNK, _H), jnp.float32),
        pltpu.VMEM((2, _CHUNK, _H), jnp.float32),
        pltpu.VMEM_SHARED((_NPC, _H), jnp.float32),
        pltpu.SemaphoreType.DMA((4,)),
        pltpu.SemaphoreType.DMA((3,)),
        pltpu.SemaphoreType.DMA((2,)),
        pltpu.SemaphoreType.DMA((3,)),
    ],
)
def _sc_layer(src_hbm, dst_hbm, e_hbm, x_hbm, out_hbm,
              src_v, dst_v, e_v, x_v, acc_sh, sem_i, sem_e, sem_x, sem_s):
    cid = lax.axis_index("c")
    sid = lax.axis_index("s")

    def issue_idx(c, s):
        pltpu.async_copy(src_hbm.at[cid, sid, c], src_v.at[s], sem_i.at[s])
        pltpu.async_copy(dst_hbm.at[cid, sid, c], dst_v.at[s], sem_i.at[s])

    def wait_idx(c, s):
        pltpu.make_async_copy(src_hbm.at[cid, sid, c], src_v.at[s],
                              sem_i.at[s]).wait()
        pltpu.make_async_copy(dst_hbm.at[cid, sid, c], dst_v.at[s],
                              sem_i.at[s]).wait()

    def issue_e(c, s):
        base = (sid * _CPW + c) * _CHUNK
        pltpu.async_copy(e_hbm.at[cid, pl.ds(base, _CHUNK), :], e_v.at[s],
                         sem_e.at[s])

    def wait_e(c, s):
        base = (sid * _CPW + c) * _CHUNK
        pltpu.make_async_copy(e_hbm.at[cid, pl.ds(base, _CHUNK), :], e_v.at[s],
                              sem_e.at[s]).wait()

    def issue_x(s3, s2):
        pltpu.async_copy(x_hbm.at[src_v.at[s3]], x_v.at[s2], sem_x.at[s2])

    def wait_x(s3, s2):
        pltpu.make_async_copy(x_hbm.at[src_v.at[s3]], x_v.at[s2],
                              sem_x.at[s2]).wait()

    def issue_scatter(s3, s4):
        pltpu.async_copy(e_v.at[s3], acc_sh.at[dst_v.at[s4]], sem_s.at[s3],
                         add=True)

    def wait_scatter(s3, s4):
        pltpu.make_async_copy(e_v.at[s3], acc_sh.at[dst_v.at[s4]],
                              sem_s.at[s3]).wait()

    z16 = jnp.zeros((16,), jnp.float32)

    @pl.loop(0, 64)
    def _zrow(r):
        for j in range(8):
            e_v[0, r, pl.ds(j * 16, 16)] = z16

    for k in range(_RPS // 64):
        r0 = sid * _RPS + k * 64
        pltpu.sync_copy(e_v.at[0, pl.ds(0, 64)],
                        acc_sh.at[pl.ds(r0, 64), :])
    plsc.subcore_barrier()

    issue_idx(0, 0)
    wait_idx(0, 0)
    issue_e(0, 0)
    issue_x(0, 0)
    issue_idx(1, 1)

    @pl.loop(0, _CPW)
    def _it(c):
        s3 = lax.rem(c, 3)
        s2 = lax.rem(c, 2)
        s4 = lax.rem(c, 4)
        p3 = lax.rem(c + 1, 3)
        p2 = lax.rem(c + 1, 2)
        p4 = lax.rem(c + 1, 4)

        @pl.when(c >= 2)
        def _():
            wait_scatter(lax.rem(c - 2, 3), lax.rem(c - 2, 4))

        @pl.when(c + 2 < _CPW)
        def _():
            issue_idx(c + 2, lax.rem(c + 2, 4))

        wait_e(c, s3)
        wait_x(s4, s2)

        @pl.loop(0, _CHUNK)
        def _crow(r):
            for j in range(8):
                sl = pl.ds(j * 16, 16)
                e_v[s3, r, sl] = jnp.maximum(
                    e_v[s3, r, sl] + x_v[s2, r, sl], 0.0)

        issue_scatter(s3, s4)

        @pl.when(c + 1 < _CPW)
        def _():
            wait_idx(c + 1, p4)
            issue_e(c + 1, p3)
            issue_x(p4, p2)

    wait_scatter((_CPW - 2) % 3, (_CPW - 2) % 4)
    wait_scatter((_CPW - 1) % 3, (_CPW - 1) % 4)
    plsc.subcore_barrier()

    for k in range(_RPS // 64):
        r0 = sid * _RPS + k * 64
        pltpu.sync_copy(acc_sh.at[pl.ds(r0, 64), :], e_v.at[0, pl.ds(0, 64)])
        pltpu.sync_copy(e_v.at[0, pl.ds(0, 64)],
                        out_hbm.at[cid, pl.ds(r0, 64), :])


def kernel(z, edge_index, edge_attr, emb, We1, be1, We2, be2,
           Wm1, bm1, Wm2, bm2, Wp1, bp1, Wp2, bp2):
    srcf = edge_index[0].astype(jnp.int32)
    dstf = edge_index[1].astype(jnp.int32)
    hi = dstf >= _NH
    cs = jnp.cumsum(hi.astype(jnp.int32))
    pos = jnp.where(hi, _CAP + cs - 1,
                    jnp.arange(_E, dtype=jnp.int32) - cs)
    src = jnp.zeros((_ET,), jnp.int32).at[pos].set(srcf)
    dst = jnp.full((_ET,), _NH, jnp.int32).at[pos].set(
        dstf - jnp.where(hi, _NH, 0))
    dd = jnp.zeros((_ET,), jnp.float32).at[pos].set(
        edge_attr.astype(jnp.float32))
    src = src.reshape(2, _NSUB, _CPW, _CHUNK)
    dst = dst.reshape(2, _NSUB, _CPW, _CHUNK)

    e = _edge_mlp(dd.reshape(_ET, 1), We1, be1.reshape(1, _H),
                  We2, be2.reshape(1, _H)).reshape(2, _CAP, _H)

    emb_pad = jnp.pad(emb, ((0, _H - emb.shape[0]), (0, 0)))
    x = _embed(z.reshape(_N, 1).astype(jnp.int32), emb_pad)

    for i in range(_L):
        agg = _sc_layer(src, dst, e, x)
        x = _node_mlp(x, agg, Wm1[i], bm1[i].reshape(1, _H),
                      Wm2[i], bm2[i].reshape(1, _H))

    return _head(x, Wp1, bp1.reshape(1, _H), Wp2, bp2.reshape(1, _H))


# speedup vs baseline: 1.9570x; 1.9570x over previous
import functools

import jax
import jax.numpy as jnp
import numpy as np
from jax import lax
from jax.experimental import pallas as pl
from jax.experimental.pallas import tpu as pltpu
from jax.experimental.pallas import tpu_sc as plsc

_N = 10000
_E = 320000
_H = 128
_L = 4
_NCENT = 32
_CUT = 6.0
_GAMMA = 10.0 / (_CUT - 0.0 + 1e-06) ** 2

_NW = 32
_CHUNK = 64
_CPW = 158
_EPW = _CHUNK * _CPW
_EP = _NW * _EPW
_NPAD = 10240

_EBLK = 2048
_NBLK = 2000


def _edge_mlp_body(d_ref, w1_ref, b1_ref, w2_ref, b2_ref, out_ref):
    centers = lax.broadcasted_iota(jnp.int32, (1, _NCENT), 1).astype(jnp.float32) * (
        _CUT / (_NCENT - 1))
    diff = d_ref[...] - centers
    rbf = jnp.exp((-_GAMMA) * diff * diff)
    h = jnp.dot(rbf, w1_ref[...], preferred_element_type=jnp.float32)
    h = h + b1_ref[...]
    h = h * jax.nn.sigmoid(h)
    e = jnp.dot(h, w2_ref[...], preferred_element_type=jnp.float32)
    out_ref[...] = e + b2_ref[...]


def _edge_mlp(d, w1, b1, w2, b2):
    grid = _EP // _EBLK
    return pl.pallas_call(
        _edge_mlp_body,
        grid=(grid,),
        in_specs=[
            pl.BlockSpec((_EBLK, 1), lambda i: (i, 0)),
            pl.BlockSpec((_NCENT, _H), lambda i: (0, 0)),
            pl.BlockSpec((1, _H), lambda i: (0, 0)),
            pl.BlockSpec((_H, _H), lambda i: (0, 0)),
            pl.BlockSpec((1, _H), lambda i: (0, 0)),
        ],
        out_specs=pl.BlockSpec((_EBLK, _H), lambda i: (i, 0)),
        out_shape=jax.ShapeDtypeStruct((_EP, _H), jnp.float32),
    )(d, w1, b1, w2, b2)


def _embed_body(z_ref, emb_ref, out_ref):
    ids = lax.broadcasted_iota(jnp.int32, (_NBLK, _H), 1)
    oh = (z_ref[...] == ids).astype(jnp.float32)
    out_ref[...] = jnp.dot(oh, emb_ref[...], preferred_element_type=jnp.float32)


def _embed(z2d, emb_pad):
    return pl.pallas_call(
        _embed_body,
        grid=(_N // _NBLK,),
        in_specs=[
            pl.BlockSpec((_NBLK, 1), lambda i: (i, 0)),
            pl.BlockSpec((_H, _H), lambda i: (0, 0)),
        ],
        out_specs=pl.BlockSpec((_NBLK, _H), lambda i: (i, 0)),
        out_shape=jax.ShapeDtypeStruct((_N, _H), jnp.float32),
    )(z2d, emb_pad)


def _node_mlp_body(x_ref, a0_ref, a1_ref, w1_ref, b1_ref, w2_ref, b2_ref, out_ref):
    h = x_ref[...] + a0_ref[...] + a1_ref[...]
    t = jnp.dot(h, w1_ref[...], preferred_element_type=jnp.float32) + b1_ref[...]
    t = t * jax.nn.sigmoid(t)
    o = jnp.dot(t, w2_ref[...], preferred_element_type=jnp.float32) + b2_ref[...]
    out_ref[...] = o * jax.nn.sigmoid(o)


def _node_mlp(x, a0, a1, w1, b1, w2, b2):
    return pl.pallas_call(
        _node_mlp_body,
        grid=(_N // _NBLK,),
        in_specs=[
            pl.BlockSpec((_NBLK, _H), lambda i: (i, 0)),
            pl.BlockSpec((_NBLK, _H), lambda i: (i, 0)),
            pl.BlockSpec((_NBLK, _H), lambda i: (i, 0)),
            pl.BlockSpec((_H, _H), lambda i: (0, 0)),
            pl.BlockSpec((1, _H), lambda i: (0, 0)),
            pl.BlockSpec((_H, _H), lambda i: (0, 0)),
            pl.BlockSpec((1, _H), lambda i: (0, 0)),
        ],
        out_specs=pl.BlockSpec((_NBLK, _H), lambda i: (i, 0)),
        out_shape=jax.ShapeDtypeStruct((_N, _H), jnp.float32),
    )(x, a0, a1, w1, b1, w2, b2)


def _head_body(x_ref, wp1_ref, bp1_ref, wp2_ref, bp2_ref, out_ref):
    g = jnp.mean(x_ref[...], axis=0, keepdims=True)
    t = jnp.dot(g, wp1_ref[...], preferred_element_type=jnp.float32) + bp1_ref[...]
    t = t * jax.nn.sigmoid(t)
    zz = jnp.dot(t, wp2_ref[...], preferred_element_type=jnp.float32) + bp2_ref[...]
    nrm = jnp.sqrt(jnp.sum(zz * zz, axis=-1, keepdims=True))
    out_ref[...] = zz / jnp.maximum(nrm, 1e-12)


def _head(x, wp1, bp1, wp2, bp2):
    return pl.pallas_call(
        _head_body,
        out_shape=jax.ShapeDtypeStruct((1, _H), jnp.float32),
    )(x, wp1, bp1, wp2, bp2)


_SC_MESH = plsc.VectorSubcoreMesh(core_axis_name="c", subcore_axis_name="s")


@functools.partial(
    pl.kernel,
    out_type=jax.ShapeDtypeStruct((2, _NPAD, _H), jnp.float32),
    mesh=_SC_MESH,
    scratch_types=[
        pltpu.VMEM((4, _CHUNK), jnp.int32),
        pltpu.VMEM((4, _CHUNK), jnp.int32),
        pltpu.VMEM((3, _CHUNK, _H), jnp.float32),
        pltpu.VMEM((2, _CHUNK, _H), jnp.float32),
        pltpu.VMEM_SHARED((_NPAD, _H), jnp.float32),
        pltpu.SemaphoreType.DMA((4,)),
        pltpu.SemaphoreType.DMA((3,)),
        pltpu.SemaphoreType.DMA((2,)),
        pltpu.SemaphoreType.DMA((3,)),
    ],
)
def _sc_layer(src_hbm, dst_hbm, e_hbm, x_hbm, out_hbm,
              src_v, dst_v, e_v, x_v, acc_sh, sem_i, sem_e, sem_x, sem_s):
    cid = lax.axis_index("c")
    sid = lax.axis_index("s")
    wid = sid * 2 + cid

    def issue_idx(c, s):
        pltpu.async_copy(src_hbm.at[wid, c], src_v.at[s], sem_i.at[s])
        pltpu.async_copy(dst_hbm.at[wid, c], dst_v.at[s], sem_i.at[s])

    def wait_idx(c, s):
        pltpu.make_async_copy(src_hbm.at[wid, c], src_v.at[s],
                              sem_i.at[s]).wait()
        pltpu.make_async_copy(dst_hbm.at[wid, c], dst_v.at[s],
                              sem_i.at[s]).wait()

    def issue_e(c, s):
        base = (wid * _CPW + c) * _CHUNK
        pltpu.async_copy(e_hbm.at[pl.ds(base, _CHUNK), :], e_v.at[s],
                         sem_e.at[s])

    def wait_e(c, s):
        base = (wid * _CPW + c) * _CHUNK
        pltpu.make_async_copy(e_hbm.at[pl.ds(base, _CHUNK), :], e_v.at[s],
                              sem_e.at[s]).wait()

    def issue_x(s3, s2):
        pltpu.async_copy(x_hbm.at[src_v.at[s3]], x_v.at[s2], sem_x.at[s2])

    def wait_x(s3, s2):
        pltpu.make_async_copy(x_hbm.at[src_v.at[s3]], x_v.at[s2],
                              sem_x.at[s2]).wait()

    def issue_scatter(s3, s4):
        pltpu.async_copy(e_v.at[s3], acc_sh.at[dst_v.at[s4]], sem_s.at[s3],
                         add=True)

    def wait_scatter(s3, s4):
        pltpu.make_async_copy(e_v.at[s3], acc_sh.at[dst_v.at[s4]],
                              sem_s.at[s3]).wait()

    z16 = jnp.zeros((16,), jnp.float32)

    @pl.loop(0, _CHUNK)
    def _zrow(r):
        for j in range(8):
            e_v[0, r, pl.ds(j * 16, 16)] = z16

    for k in range(640 // _CHUNK):
        pltpu.sync_copy(e_v.at[0],
                        acc_sh.at[pl.ds(sid * 640 + k * _CHUNK, _CHUNK), :])
    plsc.subcore_barrier()

    issue_idx(0, 0)
    wait_idx(0, 0)
    issue_e(0, 0)
    issue_x(0, 0)
    issue_idx(1, 1)

    @pl.loop(0, _CPW)
    def _it(c):
        s3 = lax.rem(c, 3)
        s2 = lax.rem(c, 2)
        s4 = lax.rem(c, 4)
        p3 = lax.rem(c + 1, 3)
        p2 = lax.rem(c + 1, 2)
        p4 = lax.rem(c + 1, 4)

        @pl.when(c >= 2)
        def _():
            wait_scatter(lax.rem(c - 2, 3), lax.rem(c - 2, 4))

        @pl.when(c + 2 < _CPW)
        def _():
            issue_idx(c + 2, lax.rem(c + 2, 4))

        wait_e(c, s3)
        wait_x(s4, s2)

        @pl.loop(0, _CHUNK)
        def _crow(r):
            for j in range(8):
                sl = pl.ds(j * 16, 16)
                e_v[s3, r, sl] = jnp.maximum(
                    e_v[s3, r, sl] + x_v[s2, r, sl], 0.0)

        issue_scatter(s3, s4)

        @pl.when(c + 1 < _CPW)
        def _():
            wait_idx(c + 1, p4)
            issue_e(c + 1, p3)
            issue_x(p4, p2)

    wait_scatter((_CPW - 2) % 3, (_CPW - 2) % 4)
    wait_scatter((_CPW - 1) % 3, (_CPW - 1) % 4)
    plsc.subcore_barrier()

    for k in range(640 // _CHUNK):
        r0 = sid * 640 + k * _CHUNK
        pltpu.sync_copy(acc_sh.at[pl.ds(r0, _CHUNK), :], e_v.at[0])
        pltpu.sync_copy(e_v.at[0], out_hbm.at[cid, pl.ds(r0, _CHUNK), :])


def kernel(z, edge_index, edge_attr, emb, We1, be1, We2, be2,
           Wm1, bm1, Wm2, bm2, Wp1, bp1, Wp2, bp2):
    src = jnp.pad(edge_index[0].astype(jnp.int32),
                  (0, _EP - _E)).reshape(_NW, _CPW, _CHUNK)
    dst = jnp.pad(edge_index[1].astype(jnp.int32), (0, _EP - _E),
                  constant_values=_N).reshape(_NW, _CPW, _CHUNK)
    d2 = jnp.pad(edge_attr.astype(jnp.float32), (0, _EP - _E)).reshape(_EP, 1)

    e = _edge_mlp(d2, We1, be1.reshape(1, _H), We2, be2.reshape(1, _H))

    emb_pad = jnp.pad(emb, ((0, _H - emb.shape[0]), (0, 0)))
    x = _embed(z.reshape(_N, 1).astype(jnp.int32), emb_pad)

    for i in range(_L):
        agg = _sc_layer(src, dst, e, x)
        x = _node_mlp(x, agg[0], agg[1], Wm1[i], bm1[i].reshape(1, _H),
                      Wm2[i], bm2[i].reshape(1, _H))

    return _head(x, Wp1, bp1.reshape(1, _H), Wp2, bp2.reshape(1, _H))


# speedup vs baseline: 2.7759x; 1.4185x over previous
import functools

import jax
import jax.numpy as jnp
import numpy as np
from jax import lax
from jax.experimental import pallas as pl
from jax.experimental.pallas import tpu as pltpu
from jax.experimental.pallas import tpu_sc as plsc

_N = 10000
_E = 320000
_H = 128
_L = 4
_NCENT = 32
_CUT = 6.0
_GAMMA = 10.0 / (_CUT - 0.0 + 1e-06) ** 2

_NW = 32
_CHUNK = 64
_CPW = 158
_EPW = _CHUNK * _CPW
_EP = _NW * _EPW
_NPAD = 10240

_EBLK = 2048
_NBLK = 2000


def _edge_mlp_body(d_ref, w1_ref, b1_ref, w2_ref, b2_ref, out_ref):
    centers = lax.broadcasted_iota(jnp.int32, (1, _NCENT), 1).astype(jnp.float32) * (
        _CUT / (_NCENT - 1))
    diff = d_ref[...] - centers
    rbf = jnp.exp((-_GAMMA) * diff * diff)
    h = jnp.dot(rbf, w1_ref[...], preferred_element_type=jnp.float32)
    h = h + b1_ref[...]
    h = h * jax.nn.sigmoid(h)
    e = jnp.dot(h, w2_ref[...], preferred_element_type=jnp.float32)
    out_ref[...] = e + b2_ref[...]


def _edge_mlp(d, w1, b1, w2, b2):
    grid = _EP // _EBLK
    return pl.pallas_call(
        _edge_mlp_body,
        grid=(grid,),
        in_specs=[
            pl.BlockSpec((_EBLK, 1), lambda i: (i, 0)),
            pl.BlockSpec((_NCENT, _H), lambda i: (0, 0)),
            pl.BlockSpec((1, _H), lambda i: (0, 0)),
            pl.BlockSpec((_H, _H), lambda i: (0, 0)),
            pl.BlockSpec((1, _H), lambda i: (0, 0)),
        ],
        out_specs=pl.BlockSpec((_EBLK, _H), lambda i: (i, 0)),
        out_shape=jax.ShapeDtypeStruct((_EP, _H), jnp.float32),
    )(d, w1, b1, w2, b2)


def _embed_body(z_ref, emb_ref, out_ref):
    ids = lax.broadcasted_iota(jnp.int32, (_NBLK, _H), 1)
    oh = (z_ref[...] == ids).astype(jnp.float32)
    out_ref[...] = jnp.dot(oh, emb_ref[...], preferred_element_type=jnp.float32)


def _embed(z2d, emb_pad):
    return pl.pallas_call(
        _embed_body,
        grid=(_N // _NBLK,),
        in_specs=[
            pl.BlockSpec((_NBLK, 1), lambda i: (i, 0)),
            pl.BlockSpec((_H, _H), lambda i: (0, 0)),
        ],
        out_specs=pl.BlockSpec((_NBLK, _H), lambda i: (i, 0)),
        out_shape=jax.ShapeDtypeStruct((_N, _H), jnp.float32),
    )(z2d, emb_pad)


def _node_mlp_body(x_ref, a0_ref, a1_ref, w1_ref, b1_ref, w2_ref, b2_ref, out_ref):
    h = x_ref[...] + a0_ref[...] + a1_ref[...]
    t = jnp.dot(h, w1_ref[...], preferred_element_type=jnp.float32) + b1_ref[...]
    t = t * jax.nn.sigmoid(t)
    o = jnp.dot(t, w2_ref[...], preferred_element_type=jnp.float32) + b2_ref[...]
    out_ref[...] = o * jax.nn.sigmoid(o)


def _node_mlp(x, a0, a1, w1, b1, w2, b2):
    return pl.pallas_call(
        _node_mlp_body,
        grid=(_N // _NBLK,),
        in_specs=[
            pl.BlockSpec((_NBLK, _H), lambda i: (i, 0)),
            pl.BlockSpec((_NBLK, _H), lambda i: (i, 0)),
            pl.BlockSpec((_NBLK, _H), lambda i: (i, 0)),
            pl.BlockSpec((_H, _H), lambda i: (0, 0)),
            pl.BlockSpec((1, _H), lambda i: (0, 0)),
            pl.BlockSpec((_H, _H), lambda i: (0, 0)),
            pl.BlockSpec((1, _H), lambda i: (0, 0)),
        ],
        out_specs=pl.BlockSpec((_NBLK, _H), lambda i: (i, 0)),
        out_shape=jax.ShapeDtypeStruct((_N, _H), jnp.float32),
    )(x, a0, a1, w1, b1, w2, b2)


def _head_body(x_ref, wp1_ref, bp1_ref, wp2_ref, bp2_ref, out_ref):
    g = jnp.mean(x_ref[...], axis=0, keepdims=True)
    t = jnp.dot(g, wp1_ref[...], preferred_element_type=jnp.float32) + bp1_ref[...]
    t = t * jax.nn.sigmoid(t)
    zz = jnp.dot(t, wp2_ref[...], preferred_element_type=jnp.float32) + bp2_ref[...]
    nrm = jnp.sqrt(jnp.sum(zz * zz, axis=-1, keepdims=True))
    out_ref[...] = zz / jnp.maximum(nrm, 1e-12)


def _head(x, wp1, bp1, wp2, bp2):
    return pl.pallas_call(
        _head_body,
        out_shape=jax.ShapeDtypeStruct((1, _H), jnp.float32),
    )(x, wp1, bp1, wp2, bp2)


_SC_MESH = plsc.VectorSubcoreMesh(core_axis_name="c", subcore_axis_name="s")


@functools.partial(
    pl.kernel,
    out_type=jax.ShapeDtypeStruct((2, _NPAD, _H), jnp.float32),
    mesh=_SC_MESH,
    scratch_types=[
        pltpu.VMEM((4, _CHUNK), jnp.int32),
        pltpu.VMEM((4, _CHUNK), jnp.int32),
        pltpu.VMEM((4, _CHUNK, _H), jnp.float32),
        pltpu.VMEM_SHARED((_NPAD, _H), jnp.float32),
        pltpu.SemaphoreType.DMA((4,)),
        pltpu.SemaphoreType.DMA((4,)),
        pltpu.SemaphoreType.DMA((2,)),
        pltpu.SemaphoreType.DMA((4,)),
    ],
)
def _sc_layer(src_hbm, dst_hbm, e_hbm, x_hbm, out_hbm,
              src_v, dst_v, e_v, acc_sh, sem_i, sem_e, sem_x, sem_s):
    cid = lax.axis_index("c")
    sid = lax.axis_index("s")
    wid = sid * 2 + cid

    def issue_idx(c, s):
        pltpu.async_copy(src_hbm.at[wid, c], src_v.at[s], sem_i.at[s])
        pltpu.async_copy(dst_hbm.at[wid, c], dst_v.at[s], sem_i.at[s])

    def wait_idx(c, s):
        pltpu.make_async_copy(src_hbm.at[wid, c], src_v.at[s],
                              sem_i.at[s]).wait()
        pltpu.make_async_copy(dst_hbm.at[wid, c], dst_v.at[s],
                              sem_i.at[s]).wait()

    def issue_e(c, s):
        base = (wid * _CPW + c) * _CHUNK
        pltpu.async_copy(e_hbm.at[pl.ds(base, _CHUNK), :], e_v.at[s],
                         sem_e.at[s])

    def wait_e(c, s):
        base = (wid * _CPW + c) * _CHUNK
        pltpu.make_async_copy(e_hbm.at[pl.ds(base, _CHUNK), :], e_v.at[s],
                              sem_e.at[s]).wait()

    def issue_x(s4, s2):
        pltpu.async_copy(x_hbm.at[src_v.at[s4]], e_v.at[s4], sem_x.at[s2],
                         add=True)

    def wait_x(s4, s2):
        pltpu.make_async_copy(x_hbm.at[src_v.at[s4]], e_v.at[s4],
                              sem_x.at[s2]).wait()

    def issue_scatter(s4):
        pltpu.async_copy(e_v.at[s4], acc_sh.at[dst_v.at[s4]], sem_s.at[s4],
                         add=True)

    def wait_scatter(s4):
        pltpu.make_async_copy(e_v.at[s4], acc_sh.at[dst_v.at[s4]],
                              sem_s.at[s4]).wait()

    z16 = jnp.zeros((16,), jnp.float32)

    @pl.loop(0, _CHUNK)
    def _zrow(r):
        for j in range(8):
            e_v[0, r, pl.ds(j * 16, 16)] = z16

    for k in range(640 // _CHUNK):
        pltpu.sync_copy(e_v.at[0],
                        acc_sh.at[pl.ds(sid * 640 + k * _CHUNK, _CHUNK), :])
    plsc.subcore_barrier()

    issue_idx(0, 0)
    issue_e(0, 0)
    issue_idx(1, 1)
    issue_e(1, 1)
    wait_e(0, 0)
    wait_idx(0, 0)
    issue_x(0, 0)

    @pl.loop(0, _CPW)
    def _it(c):
        s4 = lax.rem(c, 4)
        s2 = lax.rem(c, 2)
        p4 = lax.rem(c + 1, 4)
        p2 = lax.rem(c + 1, 2)

        @pl.when(c >= 2)
        def _():
            wait_scatter(lax.rem(c - 2, 4))

        @pl.when(c + 2 < _CPW)
        def _():
            issue_idx(c + 2, lax.rem(c + 2, 4))

        @pl.when(c + 1 < _CPW)
        def _():
            wait_e(c + 1, p4)
            wait_idx(c + 1, p4)
            issue_x(p4, p2)

        wait_x(s4, s2)

        @pl.loop(0, _CHUNK)
        def _crow(r):
            for j in range(8):
                sl = pl.ds(j * 16, 16)
                e_v[s4, r, sl] = jnp.maximum(e_v[s4, r, sl], 0.0)

        issue_scatter(s4)

        @pl.when(c + 2 < _CPW)
        def _():
            issue_e(c + 2, lax.rem(c + 2, 4))

    wait_scatter((_CPW - 2) % 4)
    wait_scatter((_CPW - 1) % 4)
    plsc.subcore_barrier()

    for k in range(640 // _CHUNK):
        r0 = sid * 640 + k * _CHUNK
        pltpu.sync_copy(acc_sh.at[pl.ds(r0, _CHUNK), :], e_v.at[0])
        pltpu.sync_copy(e_v.at[0], out_hbm.at[cid, pl.ds(r0, _CHUNK), :])


def kernel(z, edge_index, edge_attr, emb, We1, be1, We2, be2,
           Wm1, bm1, Wm2, bm2, Wp1, bp1, Wp2, bp2):
    src = jnp.pad(edge_index[0].astype(jnp.int32),
                  (0, _EP - _E)).reshape(_NW, _CPW, _CHUNK)
    dst = jnp.pad(edge_index[1].astype(jnp.int32), (0, _EP - _E),
                  constant_values=_N).reshape(_NW, _CPW, _CHUNK)
    d2 = jnp.pad(edge_attr.astype(jnp.float32), (0, _EP - _E)).reshape(_EP, 1)

    e = _edge_mlp(d2, We1, be1.reshape(1, _H), We2, be2.reshape(1, _H))

    emb_pad = jnp.pad(emb, ((0, _H - emb.shape[0]), (0, 0)))
    x = _embed(z.reshape(_N, 1).astype(jnp.int32), emb_pad)

    for i in range(_L):
        agg = _sc_layer(src, dst, e, x)
        x = _node_mlp(x, agg[0], agg[1], Wm1[i], bm1[i].reshape(1, _H),
                      Wm2[i], bm2[i].reshape(1, _H))

    return _head(x, Wp1, bp1.reshape(1, _H), Wp2, bp2.reshape(1, _H))


# speedup vs baseline: 3.2799x; 1.1816x over previous
import functools

import jax
import jax.numpy as jnp
import numpy as np
from jax import lax
from jax.experimental import pallas as pl
from jax.experimental.pallas import tpu as pltpu
from jax.experimental.pallas import tpu_sc as plsc

_N = 10000
_E = 320000
_H = 128
_L = 4
_NCENT = 32
_CUT = 6.0
_GAMMA = 10.0 / (_CUT - 0.0 + 1e-06) ** 2

_NW = 32
_CHUNK = 64
_CPW = 158
_EPW = _CHUNK * _CPW
_EP = _NW * _EPW
_NPAD = 10240

_EBLK = 2048
_NBLK = 2000


def _edge_mlp_body(d_ref, w1_ref, b1_ref, w2_ref, b2_ref, out_ref):
    centers = lax.broadcasted_iota(jnp.int32, (1, _NCENT), 1).astype(jnp.float32) * (
        _CUT / (_NCENT - 1))
    diff = d_ref[...] - centers
    rbf = jnp.exp((-_GAMMA) * diff * diff)
    h = jnp.dot(rbf, w1_ref[...], preferred_element_type=jnp.float32)
    h = h + b1_ref[...]
    h = h * jax.nn.sigmoid(h)
    e = jnp.dot(h, w2_ref[...], preferred_element_type=jnp.float32)
    out_ref[...] = e + b2_ref[...]


def _edge_mlp(d, w1, b1, w2, b2):
    grid = _EP // _EBLK
    return pl.pallas_call(
        _edge_mlp_body,
        grid=(grid,),
        in_specs=[
            pl.BlockSpec((_EBLK, 1), lambda i: (i, 0)),
            pl.BlockSpec((_NCENT, _H), lambda i: (0, 0)),
            pl.BlockSpec((1, _H), lambda i: (0, 0)),
            pl.BlockSpec((_H, _H), lambda i: (0, 0)),
            pl.BlockSpec((1, _H), lambda i: (0, 0)),
        ],
        out_specs=pl.BlockSpec((_EBLK, _H), lambda i: (i, 0)),
        out_shape=jax.ShapeDtypeStruct((_EP, _H), jnp.float32),
    )(d, w1, b1, w2, b2)


def _embed_body(z_ref, emb_ref, out_ref):
    ids = lax.broadcasted_iota(jnp.int32, (_NBLK, _H), 1)
    oh = (z_ref[...] == ids).astype(jnp.float32)
    out_ref[...] = jnp.dot(oh, emb_ref[...], preferred_element_type=jnp.float32)


def _embed(z2d, emb_pad):
    return pl.pallas_call(
        _embed_body,
        grid=(_N // _NBLK,),
        in_specs=[
            pl.BlockSpec((_NBLK, 1), lambda i: (i, 0)),
            pl.BlockSpec((_H, _H), lambda i: (0, 0)),
        ],
        out_specs=pl.BlockSpec((_NBLK, _H), lambda i: (i, 0)),
        out_shape=jax.ShapeDtypeStruct((_N, _H), jnp.float32),
    )(z2d, emb_pad)


def _node_mlp_body(x_ref, a0_ref, a1_ref, w1_ref, b1_ref, w2_ref, b2_ref, out_ref):
    h = x_ref[...] + a0_ref[...] + a1_ref[...]
    t = jnp.dot(h, w1_ref[...], preferred_element_type=jnp.float32) + b1_ref[...]
    t = t * jax.nn.sigmoid(t)
    o = jnp.dot(t, w2_ref[...], preferred_element_type=jnp.float32) + b2_ref[...]
    out_ref[...] = o * jax.nn.sigmoid(o)


def _node_mlp(x, a0, a1, w1, b1, w2, b2):
    return pl.pallas_call(
        _node_mlp_body,
        grid=(_N // _NBLK,),
        in_specs=[
            pl.BlockSpec((_NBLK, _H), lambda i: (i, 0)),
            pl.BlockSpec((_NBLK, _H), lambda i: (i, 0)),
            pl.BlockSpec((_NBLK, _H), lambda i: (i, 0)),
            pl.BlockSpec((_H, _H), lambda i: (0, 0)),
            pl.BlockSpec((1, _H), lambda i: (0, 0)),
            pl.BlockSpec((_H, _H), lambda i: (0, 0)),
            pl.BlockSpec((1, _H), lambda i: (0, 0)),
        ],
        out_specs=pl.BlockSpec((_NBLK, _H), lambda i: (i, 0)),
        out_shape=jax.ShapeDtypeStruct((_N, _H), jnp.float32),
    )(x, a0, a1, w1, b1, w2, b2)


def _head_body(x_ref, wp1_ref, bp1_ref, wp2_ref, bp2_ref, out_ref):
    g = jnp.mean(x_ref[...], axis=0, keepdims=True)
    t = jnp.dot(g, wp1_ref[...], preferred_element_type=jnp.float32) + bp1_ref[...]
    t = t * jax.nn.sigmoid(t)
    zz = jnp.dot(t, wp2_ref[...], preferred_element_type=jnp.float32) + bp2_ref[...]
    nrm = jnp.sqrt(jnp.sum(zz * zz, axis=-1, keepdims=True))
    out_ref[...] = zz / jnp.maximum(nrm, 1e-12)


def _head(x, wp1, bp1, wp2, bp2):
    return pl.pallas_call(
        _head_body,
        out_shape=jax.ShapeDtypeStruct((1, _H), jnp.float32),
    )(x, wp1, bp1, wp2, bp2)


_SC_MESH = plsc.VectorSubcoreMesh(core_axis_name="c", subcore_axis_name="s")


@functools.partial(
    pl.kernel,
    out_type=jax.ShapeDtypeStruct((2, _NPAD, _H), jnp.float32),
    mesh=_SC_MESH,
    scratch_types=[
        pltpu.VMEM((4, _CHUNK), jnp.int32),
        pltpu.VMEM((4, _CHUNK), jnp.int32),
        pltpu.VMEM((4, _CHUNK, _H), jnp.float32),
        pltpu.VMEM_SHARED((_NPAD, _H), jnp.float32),
        pltpu.SemaphoreType.DMA((4,)),
        pltpu.SemaphoreType.DMA((4,)),
        pltpu.SemaphoreType.DMA((2,)),
        pltpu.SemaphoreType.DMA((4,)),
    ],
)
def _sc_layer(src_hbm, dst_hbm, e_hbm, x_hbm, out_hbm,
              src_v, dst_v, e_v, acc_sh, sem_i, sem_e, sem_x, sem_s):
    cid = lax.axis_index("c")
    sid = lax.axis_index("s")
    wid = sid * 2 + cid

    def issue_idx(c, s):
        pltpu.async_copy(src_hbm.at[wid, c], src_v.at[s], sem_i.at[s])
        pltpu.async_copy(dst_hbm.at[wid, c], dst_v.at[s], sem_i.at[s])

    def wait_idx(c, s):
        pltpu.make_async_copy(src_hbm.at[wid, c], src_v.at[s],
                              sem_i.at[s]).wait()
        pltpu.make_async_copy(dst_hbm.at[wid, c], dst_v.at[s],
                              sem_i.at[s]).wait()

    def issue_e(c, s):
        base = (wid * _CPW + c) * _CHUNK
        pltpu.async_copy(e_hbm.at[pl.ds(base, _CHUNK), :], e_v.at[s],
                         sem_e.at[s])

    def wait_e(c, s):
        base = (wid * _CPW + c) * _CHUNK
        pltpu.make_async_copy(e_hbm.at[pl.ds(base, _CHUNK), :], e_v.at[s],
                              sem_e.at[s]).wait()

    def issue_x(s4, s2):
        pltpu.async_copy(x_hbm.at[src_v.at[s4]], e_v.at[s4], sem_x.at[s2],
                         add=True)

    def wait_x(s4, s2):
        pltpu.make_async_copy(x_hbm.at[src_v.at[s4]], e_v.at[s4],
                              sem_x.at[s2]).wait()

    def issue_scatter(s4):
        pltpu.async_copy(e_v.at[s4], acc_sh.at[dst_v.at[s4]], sem_s.at[s4],
                         add=True)

    def wait_scatter(s4):
        pltpu.make_async_copy(e_v.at[s4], acc_sh.at[dst_v.at[s4]],
                              sem_s.at[s4]).wait()

    z16 = jnp.zeros((16,), jnp.float32)

    @pl.loop(0, _CHUNK)
    def _zrow(r):
        for j in range(8):
            e_v[0, r, pl.ds(j * 16, 16)] = z16

    for k in range(640 // _CHUNK):
        pltpu.sync_copy(e_v.at[0],
                        acc_sh.at[pl.ds(sid * 640 + k * _CHUNK, _CHUNK), :])
    plsc.subcore_barrier()

    issue_idx(0, 0)
    issue_e(0, 0)
    issue_idx(1, 1)
    issue_e(1, 1)
    wait_e(0, 0)
    wait_idx(0, 0)
    issue_x(0, 0)

    @pl.loop(0, _CPW)
    def _it(c):
        s4 = lax.rem(c, 4)
        s2 = lax.rem(c, 2)
        p4 = lax.rem(c + 1, 4)
        p2 = lax.rem(c + 1, 2)

        @pl.when(c >= 2)
        def _():
            wait_scatter(lax.rem(c - 2, 4))

        @pl.when(c + 2 < _CPW)
        def _():
            issue_idx(c + 2, lax.rem(c + 2, 4))
            issue_e(c + 2, lax.rem(c + 2, 4))

        @pl.when(c + 1 < _CPW)
        def _():
            wait_e(c + 1, p4)
            wait_idx(c + 1, p4)
            issue_x(p4, p2)

        wait_x(s4, s2)

        @pl.loop(0, _CHUNK)
        def _crow(r):
            for j in range(8):
                sl = pl.ds(j * 16, 16)
                e_v[s4, r, sl] = jnp.maximum(e_v[s4, r, sl], 0.0)

        issue_scatter(s4)

    wait_scatter((_CPW - 2) % 4)
    wait_scatter((_CPW - 1) % 4)
    plsc.subcore_barrier()

    for k in range(640 // _CHUNK):
        r0 = sid * 640 + k * _CHUNK
        pltpu.sync_copy(acc_sh.at[pl.ds(r0, _CHUNK), :], e_v.at[0])
        pltpu.sync_copy(e_v.at[0], out_hbm.at[cid, pl.ds(r0, _CHUNK), :])


def kernel(z, edge_index, edge_attr, emb, We1, be1, We2, be2,
           Wm1, bm1, Wm2, bm2, Wp1, bp1, Wp2, bp2):
    src = jnp.pad(edge_index[0].astype(jnp.int32),
                  (0, _EP - _E)).reshape(_NW, _CPW, _CHUNK)
    dst = jnp.pad(edge_index[1].astype(jnp.int32), (0, _EP - _E),
                  constant_values=_N).reshape(_NW, _CPW, _CHUNK)
    d2 = jnp.pad(edge_attr.astype(jnp.float32), (0, _EP - _E)).reshape(_EP, 1)

    e = _edge_mlp(d2, We1, be1.reshape(1, _H), We2, be2.reshape(1, _H))

    emb_pad = jnp.pad(emb, ((0, _H - emb.shape[0]), (0, 0)))
    x = _embed(z.reshape(_N, 1).astype(jnp.int32), emb_pad)

    for i in range(_L):
        agg = _sc_layer(src, dst, e, x)
        x = _node_mlp(x, agg[0], agg[1], Wm1[i], bm1[i].reshape(1, _H),
                      Wm2[i], bm2[i].reshape(1, _H))

    return _head(x, Wp1, bp1.reshape(1, _H), Wp2, bp2.reshape(1, _H))
